# Initial kernel scaffold; baseline (speedup 1.0000x reference)
import jax
import jax.numpy as jnp
from jax.experimental import pallas as pl


def kernel(x, W):
    raise NotImplementedError("write your pallas kernel here")


# speedup vs baseline: 4.0887x; 4.0887x over previous
import functools

import jax
import jax.numpy as jnp
from jax import lax
from jax.experimental import pallas as pl
from jax.experimental.pallas import tpu as pltpu
from jax.experimental.pallas import tpu_sc as plsc

C_IN = 100000
D_MODEL = 64
BATCH = 4096
HIST = 50

NC = 2
NS = 16
NW = NC * NS

B = BATCH * HIST
K = 128
B_PER_W = B // NW
STEPS = B_PER_W // K


def _gather_body(x_hbm, table_hbm, out_hbm, idx_v, rows_v, gsem):
    wid = lax.axis_index("s") * NC + lax.axis_index("c")
    base = wid * B_PER_W
    pltpu.sync_copy(x_hbm.at[pl.ds(base, B_PER_W)], idx_v)

    def step(j):
        pltpu.async_copy(
            table_hbm.at[idx_v.at[pl.ds(j * K, K)]], rows_v, gsem
        ).wait()
        pltpu.sync_copy(rows_v, out_hbm.at[pl.ds(base + j * K, K)])

    lax.fori_loop(0, STEPS, lambda j, c: (step(j), c)[1], 0, unroll=False)


@jax.jit
def kernel(x, W):
    x_flat = x.reshape(B)
    mesh = plsc.VectorSubcoreMesh(core_axis_name="c", subcore_axis_name="s")
    out = pl.kernel(
        _gather_body,
        out_type=jax.ShapeDtypeStruct((B, D_MODEL), jnp.float32),
        mesh=mesh,
        scratch_types=[
            pltpu.VMEM((B_PER_W,), jnp.int32),
            pltpu.VMEM((K, D_MODEL), jnp.float32),
            pltpu.SemaphoreType.DMA,
        ],
        compiler_params=pltpu.CompilerParams(use_tc_tiling_on_sc=False),
    )(x_flat, W)
    return out.reshape(BATCH, HIST, D_MODEL)


# speedup vs baseline: 4.6197x; 1.1299x over previous
import jax
import jax.numpy as jnp
from jax import lax
from jax.experimental import pallas as pl
from jax.experimental.pallas import tpu as pltpu
from jax.experimental.pallas import tpu_sc as plsc

C_IN = 100000
D_MODEL = 64
BATCH = 4096
HIST = 50

NC = 2
NS = 16
NW = NC * NS

B = BATCH * HIST
K = 128
B_PER_W = B // NW
STEPS = B_PER_W // K
NBUF = 5
INNER = NBUF


def _gather_body(x_hbm, table_hbm, out_hbm, idx_v, rows, gsems, osems):
    wid = lax.axis_index("s") * NC + lax.axis_index("c")
    base = wid * B_PER_W
    pltpu.sync_copy(x_hbm.at[pl.ds(base, B_PER_W)], idx_v)

    def gather(j, b):
        return pltpu.make_async_copy(
            table_hbm.at[idx_v.at[pl.ds(j * K, K)]], rows[b], gsems[b]
        )

    def writeback(j, b):
        return pltpu.make_async_copy(
            rows[b], out_hbm.at[pl.ds(base + j * K, K)], osems[b]
        )

    gather(0, 0).start()

    @pl.loop(0, STEPS // INNER)
    def _outer(p):
        j0 = p * INNER
        for t in range(INNER):
            j = j0 + t
            b = t % NBUF
            nb = (t + 1) % NBUF
            if t == NBUF - 1:
                writeback(j + 1 - NBUF, nb).wait()
            else:
                @pl.when(p > 0)
                def _():
                    writeback(j + 1 - NBUF, nb).wait()

            @pl.when(j + 1 < STEPS)
            def _():
                gather(j + 1, nb).start()

            gather(j, b).wait()
            writeback(j, b).start()

    for t in range(1, NBUF):
        jt = STEPS - NBUF + t
        writeback(jt, jt % NBUF).wait()


@jax.jit
def kernel(x, W):
    x_flat = x.reshape(B)
    mesh = plsc.VectorSubcoreMesh(core_axis_name="c", subcore_axis_name="s")
    out = pl.kernel(
        _gather_body,
        out_type=jax.ShapeDtypeStruct((B, D_MODEL), jnp.float32),
        mesh=mesh,
        scratch_types=[
            pltpu.VMEM((B_PER_W,), jnp.int32),
            tuple(pltpu.VMEM((K, D_MODEL), jnp.float32) for _ in range(NBUF)),
            tuple(pltpu.SemaphoreType.DMA for _ in range(NBUF)),
            tuple(pltpu.SemaphoreType.DMA for _ in range(NBUF)),
        ],
        compiler_params=pltpu.CompilerParams(use_tc_tiling_on_sc=False),
    )(x_flat, W)
    return out.reshape(BATCH, HIST, D_MODEL)


# speedup vs baseline: 4.6681x; 1.0105x over previous
import jax
import jax.numpy as jnp
from jax import lax
from jax.experimental import pallas as pl
from jax.experimental.pallas import tpu as pltpu
from jax.experimental.pallas import tpu_sc as plsc

C_IN = 100000
D_MODEL = 64
BATCH = 4096
HIST = 50

NC = 2
NS = 16
NW = NC * NS

B = BATCH * HIST
K = 128
B_PER_W = B // NW
STEPS = B_PER_W // K
NBUF = 5
INNER = NBUF


def _gather_body(x_hbm, table_hbm, out_hbm, idx_v, rows, gsems, osems):
    wid = lax.axis_index("s") * NC + lax.axis_index("c")
    base = wid * B_PER_W
    pltpu.sync_copy(x_hbm.at[pl.ds(base, B_PER_W)], idx_v)

    def gather(j, b):
        return pltpu.make_async_copy(
            table_hbm.at[idx_v.at[pl.ds(j * K, K)]], rows[b], gsems[b]
        )

    def writeback(j, b):
        return pltpu.make_async_copy(
            rows[b], out_hbm.at[pl.ds(base + j * K, K)], osems[b]
        )

    LA = 3

    for j in range(LA):
        gather(j, j).start()

    @pl.loop(0, STEPS // INNER)
    def _outer(p):
        j0 = p * INNER
        for t in range(INNER):
            j = j0 + t
            b = t % NBUF
            nb = (t + LA) % NBUF
            if t >= NBUF - LA:
                writeback(j + LA - NBUF, nb).wait()
            else:
                @pl.when(p > 0)
                def _():
                    writeback(j + LA - NBUF, nb).wait()

            @pl.when(j + LA < STEPS)
            def _():
                gather(j + LA, nb).start()

            gather(j, b).wait()
            writeback(j, b).start()

    for t in range(NBUF - LA):
        jt = STEPS - (NBUF - LA) + t
        writeback(jt, jt % NBUF).wait()


@jax.jit
def kernel(x, W):
    x_flat = x.reshape(B)
    mesh = plsc.VectorSubcoreMesh(core_axis_name="c", subcore_axis_name="s")
    out = pl.kernel(
        _gather_body,
        out_type=jax.ShapeDtypeStruct((B, D_MODEL), jnp.float32),
        mesh=mesh,
        scratch_types=[
            pltpu.VMEM((B_PER_W,), jnp.int32),
            tuple(pltpu.VMEM((K, D_MODEL), jnp.float32) for _ in range(NBUF)),
            tuple(pltpu.SemaphoreType.DMA for _ in range(NBUF)),
            tuple(pltpu.SemaphoreType.DMA for _ in range(NBUF)),
        ],
        compiler_params=pltpu.CompilerParams(use_tc_tiling_on_sc=False),
    )(x_flat, W)
    return out.reshape(BATCH, HIST, D_MODEL)
